# baseline (speedup 1.0000x reference)
import functools

import jax
import jax.numpy as jnp
from jax import lax
from jax.experimental import pallas as pl
from jax.experimental.pallas import tpu as pltpu
from jax.experimental.pallas import tpu_sc as plsc

N = 10000
E = 320000
F = 128
R = 16
R4F = 512
NT = 4

NC, NS, L = 2, 16, 16
CH = 40
NROWS_ALL = E // CH
RPT = NROWS_ALL // NS
NBK = 20
NBLK = RPT // NBK
NROW = 640
N_PAD = NROW * NS
ZR = 40
NROW_LAST = N - NROW * (NS - 1)

BN = 2000
BE = 4000


def _node_tc_body(s_ref, v_ref, w1t_ref, b1_ref, w2t_ref, b2_ref, g_ref):
    t = pl.program_id(0)
    h = jnp.dot(s_ref[...], w1t_ref[...], preferred_element_type=jnp.float32)
    h = h + b1_ref[...]
    h = h * jax.nn.sigmoid(h)
    g = jnp.dot(h, w2t_ref[...], preferred_element_type=jnp.float32) + b2_ref[...]
    g_ref[...] = jnp.where(t == 1, g * v_ref[...], g)


def _edge_tc_body(r1_ref, r2_ref, fc1_ref, fc2_ref, u1_ref, u2_ref, wrt_ref,
                  br_ref, w_ref):
    t = pl.program_id(0)
    fc1 = fc1_ref[...]
    fc2 = fc2_ref[...]
    a = fc1 * r1_ref[...] + fc2 * r2_ref[...]
    w = jnp.dot(a, wrt_ref[...], preferred_element_type=jnp.float32)
    w = w + (fc1 + fc2) * br_ref[...]
    factor = jnp.where(t == 2, u1_ref[...], jnp.where(t == 3, u2_ref[...], 1.0))
    w_ref[...] = w * factor


def _combine_tc_body(s_ref, v_ref, p_ref, os_ref, ov_ref):
    os_ref[...] = s_ref[...] + p_ref[0]
    ov_ref[...] = v_ref[...] + (p_ref[1] + p_ref[2] + p_ref[3])


_sc_mesh = plsc.VectorSubcoreMesh(core_axis_name="c", subcore_axis_name="s")


@functools.partial(
    pl.kernel,
    out_type=jax.ShapeDtypeStruct((NT, N, F), jnp.float32),
    mesh=_sc_mesh,
    scratch_types=[
        pltpu.VMEM((NBK, CH), jnp.int32),
        pltpu.VMEM((NBK, CH), jnp.int32),
        pltpu.VMEM((CH, F), jnp.float32),
        pltpu.VMEM((CH, F), jnp.float32),
        pltpu.VMEM((CH, F), jnp.float32),
        pltpu.VMEM((CH, F), jnp.float32),
        pltpu.VMEM((CH, F), jnp.float32),
        pltpu.VMEM((CH, F), jnp.float32),
        pltpu.VMEM((ZR, F), jnp.float32),
        pltpu.VMEM_SHARED((N_PAD, F), jnp.float32),
        pltpu.SemaphoreType.DMA,
        pltpu.SemaphoreType.DMA,
        pltpu.SemaphoreType.DMA,
        pltpu.SemaphoreType.DMA,
    ],
)
def _sc_scatter4(g_hbm, w_hbm, idxj_hbm, idxi_hbm, out_hbm,
                 idxj_blk, idxi_blk, g_a, g_b, w_a, w_b, m_a, m_b, z_v, acc,
                 sem_ga, sem_gb, sem_wa, sem_wb):
    cid = lax.axis_index("c")
    sid = lax.axis_index("s")
    nbase = pl.multiple_of(sid * NROW, 8)

    zero = jnp.zeros((L,), jnp.float32)

    def zrow(rr, carry):
        for k in range(F // L):
            z_v[rr, pl.ds(k * L, L)] = zero
        return carry

    lax.fori_loop(0, ZR, zrow, 0)

    def zero_acc():
        for q in range(NROW // ZR):
            pltpu.sync_copy(z_v, acc.at[pl.ds(nbase + q * ZR, ZR)])

    def compute(g_v, w_v, m_v):
        def edge(c, icarry):
            for k in range(F // L):
                m_v[c, pl.ds(k * L, L)] = (
                    g_v[c, pl.ds(k * L, L)] * w_v[c, pl.ds(k * L, L)])
            return icarry

        lax.fori_loop(0, CH, edge, 0)

    def run_task(tid):
        rbase = sid * RPT

        def issue(row, ebase, g_v, w_v, sem_g, sem_w):
            e0 = pl.multiple_of(ebase, 8)
            gd = pltpu.async_copy(g_hbm.at[idxj_blk.at[row]], g_v, sem_g)
            wd = pltpu.async_copy(w_hbm.at[pl.ds(e0, CH)], w_v, sem_w)
            return gd, wd

        def wait(g_v, w_v, sem_g, sem_w):
            pltpu.make_async_copy(g_hbm.at[pl.ds(0, CH)], g_v, sem_g).wait()
            pltpu.make_async_copy(w_hbm.at[pl.ds(0, CH)], w_v, sem_w).wait()

        def block(blk, carry):
            q0 = rbase + blk * NBK
            pltpu.sync_copy(idxj_hbm.at[tid, sid, blk], idxj_blk)
            pltpu.sync_copy(idxi_hbm.at[sid, blk], idxi_blk)
            ebase0 = tid * E + q0 * CH
            issue(0, ebase0, g_a, w_a, sem_ga, sem_wa)

            def pair(k, icarry):
                r0 = 2 * k
                eb0 = ebase0 + r0 * CH
                issue(r0 + 1, eb0 + CH, g_b, w_b, sem_gb, sem_wb)
                wait(g_a, w_a, sem_ga, sem_wa)
                compute(g_a, w_a, m_a)
                pltpu.sync_copy(m_a, acc.at[idxi_blk.at[r0]], add=True)

                @pl.when(k < NBK // 2 - 1)
                def _prefetch():
                    issue(r0 + 2, eb0 + 2 * CH, g_a, w_a, sem_ga, sem_wa)

                wait(g_b, w_b, sem_gb, sem_wb)
                compute(g_b, w_b, m_b)
                pltpu.sync_copy(m_b, acc.at[idxi_blk.at[r0 + 1]], add=True)
                return icarry

            lax.fori_loop(0, NBK // 2, pair, 0)
            return carry

        lax.fori_loop(0, NBLK, block, 0)

    def copy_out(tid):
        @pl.when(sid != NS - 1)
        def _copy_full():
            pltpu.sync_copy(acc.at[pl.ds(nbase, NROW)],
                            out_hbm.at[tid, pl.ds(nbase, NROW)])

        @pl.when(sid == NS - 1)
        def _copy_tail():
            pltpu.sync_copy(acc.at[pl.ds(nbase, NROW_LAST)],
                            out_hbm.at[tid, pl.ds(nbase, NROW_LAST)])

    for q in range(NT // NC):
        tid = cid * (NT // NC) + q
        zero_acc()
        plsc.subcore_barrier()
        run_task(tid)
        plsc.subcore_barrier()
        copy_out(tid)


def kernel(s, v, radial_embeddings_1, radial_embeddings_2, f_cut_1, f_cut_2,
           unit_vectors_1, unit_vectors_2, edge_index, W1, b1, W2, b2, Wr, br):
    idx_i = edge_index[0].astype(jnp.int32)
    idx_j = edge_index[1].astype(jnp.int32)
    idxi2d = idx_i.reshape(NS, NBLK, NBK, CH)
    idxj4 = (idx_j.reshape(NROWS_ALL, CH)[None]
             + (jnp.arange(NT, dtype=jnp.int32) * N)[:, None, None]
             ).reshape(NT, NS, NBLK, NBK, CH)
    fc1 = f_cut_1.reshape(E, 1)
    fc2 = f_cut_2.reshape(E, 1)
    u1 = unit_vectors_1.reshape(E, 1)
    u2 = unit_vectors_2.reshape(E, 1)

    g4 = pl.pallas_call(
        _node_tc_body,
        grid=(NT, N // BN),
        in_specs=[
            pl.BlockSpec((BN, F), lambda t, i: (i, 0)),
            pl.BlockSpec((BN, F), lambda t, i: (i, 0)),
            pl.BlockSpec((F, F), lambda t, i: (0, 0)),
            pl.BlockSpec((1, F), lambda t, i: (0, 0)),
            pl.BlockSpec((F, F), lambda t, i: (0, t)),
            pl.BlockSpec((1, F), lambda t, i: (0, t)),
        ],
        out_specs=pl.BlockSpec((BN, F), lambda t, i: (t * (N // BN) + i, 0)),
        out_shape=jax.ShapeDtypeStruct((NT * N, F), jnp.float32),
    )(s, v, W1.T, b1.reshape(1, F), W2.T, b2.reshape(1, R4F))

    w4 = pl.pallas_call(
        _edge_tc_body,
        grid=(NT, E // BE),
        in_specs=[
            pl.BlockSpec((BE, R), lambda t, i: (i, 0)),
            pl.BlockSpec((BE, R), lambda t, i: (i, 0)),
            pl.BlockSpec((BE, 1), lambda t, i: (i, 0)),
            pl.BlockSpec((BE, 1), lambda t, i: (i, 0)),
            pl.BlockSpec((BE, 1), lambda t, i: (i, 0)),
            pl.BlockSpec((BE, 1), lambda t, i: (i, 0)),
            pl.BlockSpec((R, F), lambda t, i: (0, t)),
            pl.BlockSpec((1, F), lambda t, i: (0, t)),
        ],
        out_specs=pl.BlockSpec((BE, F), lambda t, i: (t * (E // BE) + i, 0)),
        out_shape=jax.ShapeDtypeStruct((NT * E, F), jnp.float32),
    )(radial_embeddings_1, radial_embeddings_2, fc1, fc2, u1, u2,
      Wr.T, br.reshape(1, R4F))

    p4 = _sc_scatter4(g4, w4, idxj4, idxi2d)

    out_s, out_v = pl.pallas_call(
        _combine_tc_body,
        grid=(N // BN,),
        in_specs=[
            pl.BlockSpec((BN, F), lambda i: (i, 0)),
            pl.BlockSpec((BN, F), lambda i: (i, 0)),
            pl.BlockSpec((NT, BN, F), lambda i: (0, i, 0)),
        ],
        out_specs=[
            pl.BlockSpec((BN, F), lambda i: (i, 0)),
            pl.BlockSpec((BN, F), lambda i: (i, 0)),
        ],
        out_shape=[
            jax.ShapeDtypeStruct((N, F), jnp.float32),
            jax.ShapeDtypeStruct((N, F), jnp.float32),
        ],
    )(s, v, p4)

    return out_s, out_v
